# Initial kernel scaffold; baseline (speedup 1.0000x reference)
import jax
import jax.numpy as jnp
from jax.experimental import pallas as pl


def kernel(x, edge_index, W1, a_src1, a_dst1, b1, ca1_w1, ca1_b1, ca1_w2, ca1_b2, W2, a_src2, a_dst2, b2, ca2_w1, ca2_b1, ca2_w2, ca2_b2, res_W, res_b, fc_W, fc_b):
    raise NotImplementedError("write your pallas kernel here")


# speedup vs baseline: 131.1265x; 131.1265x over previous
import functools

import jax
import jax.numpy as jnp
from jax import lax
from jax.experimental import pallas as pl
from jax.experimental.pallas import tpu as pltpu
from jax.experimental.pallas import tpu_sc as plsc

_NC = 2
_NS = 16
_NW = _NC * _NS
_L = 16


def _sc_edge_pass(table, src, dst):
    n = table.shape[1]
    e = src.shape[0]
    epw = e // _NW
    steps = epw // _L

    def body(tbl_hbm, src_hbm, dst_hbm, out_hbm, tbl_v, acc_v, src_v, dst_v, sem):
        wid = lax.axis_index("s") * _NC + lax.axis_index("c")
        base = wid * epw
        cp_t = pltpu.async_copy(tbl_hbm, tbl_v, sem)
        cp_s = pltpu.async_copy(src_hbm.at[pl.ds(base, epw)], src_v, sem)
        cp_d = pltpu.async_copy(dst_hbm.at[pl.ds(base, epw)], dst_v, sem)

        zero = jnp.zeros((_L,), jnp.float32)

        def zbody(i, carry):
            acc_v[pl.ds(i * _L, _L)] = zero
            return carry

        lax.fori_loop(0, (5 * n) // _L, zbody, 0)
        cp_t.wait()
        cp_s.wait()
        cp_d.wait()

        rows = [jnp.full((_L,), r * n, jnp.int32) for r in range(6)]

        def ebody(i, carry):
            off = i * _L
            sidx = src_v[pl.ds(off, _L)]
            didx = dst_v[pl.ds(off, _L)]
            av = plsc.load_gather(tbl_v, [sidx + rows[4]])
            dv = plsc.load_gather(tbl_v, [didx + rows[5]])
            s = av + dv
            w = jnp.exp(jnp.maximum(s, 0.2 * s))
            plsc.addupdate_scatter(acc_v, [didx + rows[4]], w)
            for r in range(4):
                h = plsc.load_gather(tbl_v, [sidx + rows[r]])
                plsc.addupdate_scatter(acc_v, [didx + rows[r]], h * w)
            return carry

        lax.fori_loop(0, steps, ebody, 0)
        pltpu.sync_copy(acc_v, out_hbm.at[wid])

    return pl.kernel(
        body,
        out_type=jax.ShapeDtypeStruct((_NW, 5 * n), jnp.float32),
        mesh=plsc.VectorSubcoreMesh(
            core_axis_name="c", subcore_axis_name="s",
            num_cores=_NC, num_subcores=_NS),
        compiler_params=pltpu.CompilerParams(needs_layout_passes=False),
        scratch_types=[
            pltpu.VMEM((6 * n,), jnp.float32),
            pltpu.VMEM((5 * n,), jnp.float32),
            pltpu.VMEM((epw,), jnp.int32),
            pltpu.VMEM((epw,), jnp.int32),
            pltpu.SemaphoreType.DMA,
        ],
    )(table.reshape(-1), src, dst).reshape(_NW, 5, n)


def _identity4():
    r = lax.broadcasted_iota(jnp.int32, (4, 4), 0)
    c = lax.broadcasted_iota(jnp.int32, (4, 4), 1)
    return (r == c).astype(jnp.float32)


def _channel_attention_t(o, w1, b1c, w2, b2c):
    i4 = _identity4()
    m = jnp.sum(o, axis=1, keepdims=True) * (1.0 / o.shape[1])
    s_row = jnp.sum(m * w1, axis=0, keepdims=True)
    s_col = jnp.sum(s_row * i4, axis=1, keepdims=True)
    s_col = jnp.maximum(s_col + b1c, 0.0)
    g_row = jnp.sum(s_col * w2, axis=0, keepdims=True)
    g_col = jnp.sum(g_row * i4, axis=1, keepdims=True) + b2c
    return o * (1.0 / (1.0 + jnp.exp(-g_col)))


def _tc_pre(x, pt, a_s, a_d, res_b):
    n = x.shape[0]

    def body(x_ref, pt_ref, as_ref, ad_ref, rb_ref, tbl_ref, res_ref):
        y = lax.dot_general(pt_ref[...], x_ref[...], (((1,), (1,)), ((), ())),
                            preferred_element_type=jnp.float32)
        h = y[0:4]
        asr = jnp.sum(h * as_ref[...], axis=0, keepdims=True)
        adr = jnp.sum(h * ad_ref[...], axis=0, keepdims=True)
        tbl_ref[...] = jnp.concatenate([h, asr, adr], axis=0)
        res_ref[...] = y[4:8] + rb_ref[...]

    return pl.pallas_call(
        body,
        out_shape=(jax.ShapeDtypeStruct((6, n), jnp.float32),
                   jax.ShapeDtypeStruct((4, n), jnp.float32)),
    )(x, pt, a_s, a_d, res_b)


def _reduce_norm(acc_ref, bias_c):
    a = acc_ref[0]
    for i in range(1, _NW):
        a = a + acc_ref[i]
    o = a[0:4] / (a[4:5] + 1e-16) + bias_c
    return jnp.maximum(o, 0.0)


def _tc_mid(acc, b1c, cw1, cb1c, cw2, cb2c, w2t, as2, ad2):
    n = acc.shape[2]

    def body(acc_ref, b1_ref, w1_ref, bb1_ref, w2_ref, bb2_ref, w2t_ref,
             as_ref, ad_ref, tbl_ref):
        o = _reduce_norm(acc_ref, b1_ref[...])
        hca = _channel_attention_t(o, w1_ref[...], bb1_ref[...],
                                   w2_ref[...], bb2_ref[...])
        h2 = lax.dot_general(w2t_ref[...], hca, (((1,), (0,)), ((), ())),
                             preferred_element_type=jnp.float32)
        asr = jnp.sum(h2 * as_ref[...], axis=0, keepdims=True)
        adr = jnp.sum(h2 * ad_ref[...], axis=0, keepdims=True)
        tbl_ref[...] = jnp.concatenate([h2, asr, adr], axis=0)

    return pl.pallas_call(
        body,
        out_shape=jax.ShapeDtypeStruct((6, n), jnp.float32),
    )(acc, b1c, cw1, cb1c, cw2, cb2c, w2t, as2, ad2)


def _tc_post(acc, b2c, cw1, cb1c, cw2, cb2c, resid, fc_c, fcb):
    n = acc.shape[2]

    def body(acc_ref, b2_ref, w1_ref, bb1_ref, w2_ref, bb2_ref, res_ref,
             fc_ref, fcb_ref, out_ref):
        o = _reduce_norm(acc_ref, b2_ref[...])
        hca = _channel_attention_t(o, w1_ref[...], bb1_ref[...],
                                   w2_ref[...], bb2_ref[...])
        f = hca + res_ref[...]
        logit = jnp.sum(f * fc_ref[...], axis=0, keepdims=True) + fcb_ref[...]
        out_ref[...] = 1.0 / (1.0 + jnp.exp(-logit))

    return pl.pallas_call(
        body,
        out_shape=jax.ShapeDtypeStruct((1, n), jnp.float32),
    )(acc, b2c, cw1, cb1c, cw2, cb2c, resid, fc_c, fcb)


def kernel(x, edge_index, W1, a_src1, a_dst1, b1, ca1_w1, ca1_b1, ca1_w2,
           ca1_b2, W2, a_src2, a_dst2, b2, ca2_w1, ca2_b1, ca2_w2, ca2_b2,
           res_W, res_b, fc_W, fc_b):
    src = edge_index[0]
    dst = edge_index[1]
    pt = jnp.concatenate([W1.T, res_W.T], axis=0)

    tbl1, resid = _tc_pre(x, pt, a_src1.reshape(4, 1), a_dst1.reshape(4, 1),
                          res_b.reshape(4, 1))
    acc1 = _sc_edge_pass(tbl1, src, dst)
    tbl2 = _tc_mid(acc1, b1.reshape(4, 1), ca1_w1, ca1_b1.reshape(4, 1),
                   ca1_w2, ca1_b2.reshape(4, 1), W2.T,
                   a_src2.reshape(4, 1), a_dst2.reshape(4, 1))
    acc2 = _sc_edge_pass(tbl2, src, dst)
    out = _tc_post(acc2, b2.reshape(4, 1), ca2_w1, ca2_b1.reshape(4, 1),
                   ca2_w2, ca2_b2.reshape(4, 1), resid,
                   fc_W.reshape(4, 1), fc_b.reshape(1, 1))
    return out.reshape(-1, 1)


# speedup vs baseline: 183.2178x; 1.3973x over previous
import functools

import jax
import jax.numpy as jnp
from jax import lax
from jax.experimental import pallas as pl
from jax.experimental.pallas import tpu as pltpu
from jax.experimental.pallas import tpu_sc as plsc

_NC = 2
_NS = 16
_NW = _NC * _NS
_L = 16


def _sc_edge_pass(table, src, dst):
    n = table.shape[1]
    e = src.shape[0]
    epw = e // _NW
    steps = epw // _L

    def body(tbl_hbm, src_hbm, dst_hbm, out_hbm, tbl_v, acc_v, src_v, dst_v, sem):
        wid = lax.axis_index("s") * _NC + lax.axis_index("c")
        base = wid * epw
        cp_t = pltpu.async_copy(tbl_hbm, tbl_v, sem)
        cp_s = pltpu.async_copy(src_hbm.at[pl.ds(base, epw)], src_v, sem)
        cp_d = pltpu.async_copy(dst_hbm.at[pl.ds(base, epw)], dst_v, sem)

        zero = jnp.zeros((_L,), jnp.float32)

        @plsc.parallel_loop(0, 5 * n, step=_L, unroll=8)
        def _zero(off):
            acc_v[pl.ds(off, _L)] = zero

        cp_t.wait()
        cp_s.wait()
        cp_d.wait()

        rows = [jnp.full((_L,), r * n, jnp.int32) for r in range(6)]

        @plsc.parallel_loop(0, epw, step=_L, unroll=4)
        def _edges(off):
            sidx = src_v[pl.ds(off, _L)]
            didx = dst_v[pl.ds(off, _L)]
            av = plsc.load_gather(tbl_v, [sidx + rows[4]])
            dv = plsc.load_gather(tbl_v, [didx + rows[5]])
            s = av + dv
            w = jnp.exp(jnp.maximum(s, 0.2 * s))
            plsc.addupdate_scatter(acc_v, [didx + rows[4]], w)
            for r in range(4):
                h = plsc.load_gather(tbl_v, [sidx + rows[r]])
                plsc.addupdate_scatter(acc_v, [didx + rows[r]], h * w)

        pltpu.sync_copy(acc_v, out_hbm.at[wid])

    return pl.kernel(
        body,
        out_type=jax.ShapeDtypeStruct((_NW, 5 * n), jnp.float32),
        mesh=plsc.VectorSubcoreMesh(
            core_axis_name="c", subcore_axis_name="s",
            num_cores=_NC, num_subcores=_NS),
        compiler_params=pltpu.CompilerParams(needs_layout_passes=False),
        scratch_types=[
            pltpu.VMEM((6 * n,), jnp.float32),
            pltpu.VMEM((5 * n,), jnp.float32),
            pltpu.VMEM((epw,), jnp.int32),
            pltpu.VMEM((epw,), jnp.int32),
            pltpu.SemaphoreType.DMA,
        ],
    )(table.reshape(-1), src, dst).reshape(_NW, 5, n)


def _identity4():
    r = lax.broadcasted_iota(jnp.int32, (4, 4), 0)
    c = lax.broadcasted_iota(jnp.int32, (4, 4), 1)
    return (r == c).astype(jnp.float32)


def _channel_attention_t(o, w1, b1c, w2, b2c):
    i4 = _identity4()
    m = jnp.sum(o, axis=1, keepdims=True) * (1.0 / o.shape[1])
    s_row = jnp.sum(m * w1, axis=0, keepdims=True)
    s_col = jnp.sum(s_row * i4, axis=1, keepdims=True)
    s_col = jnp.maximum(s_col + b1c, 0.0)
    g_row = jnp.sum(s_col * w2, axis=0, keepdims=True)
    g_col = jnp.sum(g_row * i4, axis=1, keepdims=True) + b2c
    return o * (1.0 / (1.0 + jnp.exp(-g_col)))


def _tc_pre(x, pt, a_s, a_d, res_b):
    n = x.shape[0]

    def body(x_ref, pt_ref, as_ref, ad_ref, rb_ref, tbl_ref, res_ref):
        y = lax.dot_general(pt_ref[...], x_ref[...], (((1,), (1,)), ((), ())),
                            preferred_element_type=jnp.float32)
        h = y[0:4]
        asr = jnp.sum(h * as_ref[...], axis=0, keepdims=True)
        adr = jnp.sum(h * ad_ref[...], axis=0, keepdims=True)
        tbl_ref[...] = jnp.concatenate([h, asr, adr], axis=0)
        res_ref[...] = y[4:8] + rb_ref[...]

    return pl.pallas_call(
        body,
        out_shape=(jax.ShapeDtypeStruct((6, n), jnp.float32),
                   jax.ShapeDtypeStruct((4, n), jnp.float32)),
    )(x, pt, a_s, a_d, res_b)


def _reduce_norm(acc_ref, bias_c):
    a = acc_ref[0]
    for i in range(1, _NW):
        a = a + acc_ref[i]
    o = a[0:4] / (a[4:5] + 1e-16) + bias_c
    return jnp.maximum(o, 0.0)


def _tc_mid(acc, b1c, cw1, cb1c, cw2, cb2c, w2t, as2, ad2):
    n = acc.shape[2]

    def body(acc_ref, b1_ref, w1_ref, bb1_ref, w2_ref, bb2_ref, w2t_ref,
             as_ref, ad_ref, tbl_ref):
        o = _reduce_norm(acc_ref, b1_ref[...])
        hca = _channel_attention_t(o, w1_ref[...], bb1_ref[...],
                                   w2_ref[...], bb2_ref[...])
        h2 = lax.dot_general(w2t_ref[...], hca, (((1,), (0,)), ((), ())),
                             preferred_element_type=jnp.float32)
        asr = jnp.sum(h2 * as_ref[...], axis=0, keepdims=True)
        adr = jnp.sum(h2 * ad_ref[...], axis=0, keepdims=True)
        tbl_ref[...] = jnp.concatenate([h2, asr, adr], axis=0)

    return pl.pallas_call(
        body,
        out_shape=jax.ShapeDtypeStruct((6, n), jnp.float32),
    )(acc, b1c, cw1, cb1c, cw2, cb2c, w2t, as2, ad2)


def _tc_post(acc, b2c, cw1, cb1c, cw2, cb2c, resid, fc_c, fcb):
    n = acc.shape[2]

    def body(acc_ref, b2_ref, w1_ref, bb1_ref, w2_ref, bb2_ref, res_ref,
             fc_ref, fcb_ref, out_ref):
        o = _reduce_norm(acc_ref, b2_ref[...])
        hca = _channel_attention_t(o, w1_ref[...], bb1_ref[...],
                                   w2_ref[...], bb2_ref[...])
        f = hca + res_ref[...]
        logit = jnp.sum(f * fc_ref[...], axis=0, keepdims=True) + fcb_ref[...]
        out_ref[...] = 1.0 / (1.0 + jnp.exp(-logit))

    return pl.pallas_call(
        body,
        out_shape=jax.ShapeDtypeStruct((1, n), jnp.float32),
    )(acc, b2c, cw1, cb1c, cw2, cb2c, resid, fc_c, fcb)


def kernel(x, edge_index, W1, a_src1, a_dst1, b1, ca1_w1, ca1_b1, ca1_w2,
           ca1_b2, W2, a_src2, a_dst2, b2, ca2_w1, ca2_b1, ca2_w2, ca2_b2,
           res_W, res_b, fc_W, fc_b):
    src = edge_index[0]
    dst = edge_index[1]
    pt = jnp.concatenate([W1.T, res_W.T], axis=0)

    tbl1, resid = _tc_pre(x, pt, a_src1.reshape(4, 1), a_dst1.reshape(4, 1),
                          res_b.reshape(4, 1))
    acc1 = _sc_edge_pass(tbl1, src, dst)
    tbl2 = _tc_mid(acc1, b1.reshape(4, 1), ca1_w1, ca1_b1.reshape(4, 1),
                   ca1_w2, ca1_b2.reshape(4, 1), W2.T,
                   a_src2.reshape(4, 1), a_dst2.reshape(4, 1))
    acc2 = _sc_edge_pass(tbl2, src, dst)
    out = _tc_post(acc2, b2.reshape(4, 1), ca2_w1, ca2_b1.reshape(4, 1),
                   ca2_w2, ca2_b2.reshape(4, 1), resid,
                   fc_W.reshape(4, 1), fc_b.reshape(1, 1))
    return out.reshape(-1, 1)
